# Initial kernel scaffold; baseline (speedup 1.0000x reference)
import jax
import jax.numpy as jnp
from jax.experimental import pallas as pl


def kernel(x, router_weight, router_bias, gate_up_proj, gate_up_proj_bias, down_proj, down_proj_bias):
    raise NotImplementedError("write your pallas kernel here")


# speedup vs baseline: 4.3997x; 4.3997x over previous
import functools

import jax
import jax.numpy as jnp
from jax.experimental import pallas as pl
from jax.experimental.pallas import tpu as pltpu
from jax.experimental.pallas import tpu_sc as plsc

NE = 64
K = 2
D = 768
F = 768
T = 2048
NPAIR = T * K
BLK = 128
NBLK = NPAIR // BLK + NE
PADTOT = NBLK * BLK
ALPHA = 1.702
LIMIT = 7.0

_VMESH = functools.partial(
    plsc.VectorSubcoreMesh, core_axis_name="core", subcore_axis_name="subcore"
)


def _router_body(x_ref, rw_ref, rb_ref, ids_ref, ew_ref):
    x = x_ref[...]
    rw = rw_ref[...]
    logits = jax.lax.dot_general(
        x, rw, (((1,), (1,)), ((), ())),
        preferred_element_type=jnp.float32,
        precision=jax.lax.Precision.DEFAULT,
    ) + rb_ref[...]
    iota = jax.lax.broadcasted_iota(jnp.int32, logits.shape, 1)
    m1 = jnp.max(logits, axis=1, keepdims=True)
    a1 = jnp.min(jnp.where(logits == m1, iota, NE), axis=1, keepdims=True)
    l2 = jnp.where(iota == a1, -jnp.inf, logits)
    m2 = jnp.max(l2, axis=1, keepdims=True)
    a2 = jnp.min(jnp.where(l2 == m2, iota, NE), axis=1, keepdims=True)
    z = jnp.exp(m2 - m1)
    w1 = 1.0 / (1.0 + z)
    w2 = z / (1.0 + z)
    ids_ref[...] = jnp.concatenate([a1, a2], axis=1)
    ew_ref[...] = jnp.concatenate([w1, w2], axis=1)


def _router(x, rw, rb, interpret=False):
    return pl.pallas_call(
        _router_body,
        out_shape=[
            jax.ShapeDtypeStruct((T, K), jnp.int32),
            jax.ShapeDtypeStruct((T, K), jnp.float32),
        ],
        interpret=interpret,
    )(x, rw, rb.reshape(1, NE))


def _mlp_body(be_ref, xs_ref, gup_ref, gupb_ref, dp_ref, dpb_ref, wrow_ref,
              sel_ref, ys_ref):
    x = xs_ref[...].astype(jnp.float32)
    gu = jnp.dot(x, gup_ref[0], preferred_element_type=jnp.float32)
    gu = gu + gupb_ref[0]
    gmin = jnp.minimum(gu, LIMIT)
    glu = gmin * jax.nn.sigmoid(gmin * ALPHA)
    upc = jnp.clip(gu, -LIMIT, LIMIT) + 1.0
    up_on_gate = jnp.roll(upc, -1, axis=1)
    lane = jax.lax.broadcasted_iota(jnp.int32, gu.shape, 1)
    act2 = jnp.where(lane % 2 == 0, glu * up_on_gate, 0.0)
    act = jnp.dot(act2, sel_ref[...], preferred_element_type=jnp.float32)
    y = jnp.dot(act, dp_ref[0], preferred_element_type=jnp.float32)
    y = (y + dpb_ref[0]) * wrow_ref[...]
    ys_ref[...] = y.astype(jnp.bfloat16)


def _grouped_mlp(bexp, xs, gup, gupb, dp, dpb, wrow, interpret=False):
    grid_spec = pltpu.PrefetchScalarGridSpec(
        num_scalar_prefetch=1,
        grid=(NBLK,),
        in_specs=[
            pl.BlockSpec((BLK, D), lambda b, be: (b, 0)),
            pl.BlockSpec((1, D, 2 * F), lambda b, be: (be[b], 0, 0)),
            pl.BlockSpec((1, 1, 2 * F), lambda b, be: (be[b], 0, 0)),
            pl.BlockSpec((1, F, D), lambda b, be: (be[b], 0, 0)),
            pl.BlockSpec((1, 1, D), lambda b, be: (be[b], 0, 0)),
            pl.BlockSpec((BLK, 1), lambda b, be: (b, 0)),
            pl.BlockSpec((2 * F, F), lambda b, be: (0, 0)),
        ],
        out_specs=pl.BlockSpec((BLK, D), lambda b, be: (b, 0)),
    )
    sel = (jnp.arange(2 * F, dtype=jnp.int32)[:, None]
           == 2 * jnp.arange(F, dtype=jnp.int32)[None, :]).astype(jnp.float32)
    return pl.pallas_call(
        _mlp_body,
        grid_spec=grid_spec,
        out_shape=jax.ShapeDtypeStruct((PADTOT, D), jnp.bfloat16),
        interpret=interpret,
    )(bexp, xs, gup, gupb.reshape(NE, 1, 2 * F), dp, dpb.reshape(NE, 1, D),
      wrow, sel)


def _sc_gather(src, idx, window=128):
    n = idx.shape[0]
    d = src.shape[1]
    idx2 = idx.reshape(1, n)

    @pl.kernel(
        out_type=jax.ShapeDtypeStruct((n, d), src.dtype),
        mesh=_VMESH(),
    )
    def k(x_hbm, i_hbm, o_hbm):
        def body(i_vmem, o_vmem):
            pltpu.sync_copy(x_hbm.at[i_vmem.at[0]], o_vmem)

        pltpu.emit_pipeline(
            body,
            grid=(n // window,),
            in_specs=[pl.BlockSpec((1, window), index_map=lambda i: (0, i))],
            out_specs=[pl.BlockSpec((window, d), index_map=lambda i: (i, 0))],
            core_axis_name=("core", "subcore"),
            dimension_semantics=(pltpu.PARALLEL,),
        )(i_hbm, o_hbm)

    return k(src, idx2)


def _add_halves_body(a_ref, b_ref, o_ref):
    o_ref[...] = a_ref[...].astype(jnp.float32) + b_ref[...].astype(jnp.float32)


def _add_halves(g, interpret=False):
    nb = T // 256
    return pl.pallas_call(
        _add_halves_body,
        grid=(nb,),
        in_specs=[
            pl.BlockSpec((256, D), lambda i: (i, 0)),
            pl.BlockSpec((256, D), lambda i: (i + nb, 0)),
        ],
        out_specs=pl.BlockSpec((256, D), lambda i: (i, 0)),
        out_shape=jax.ShapeDtypeStruct((T, D), jnp.float32),
        interpret=interpret,
    )(g, g)


def _routing_metadata(ids, ew):
    ids_flat = ids.reshape(-1)
    perm = jnp.argsort(ids_flat, stable=True).astype(jnp.int32)
    se = ids_flat[perm]
    stok = (perm // K).astype(jnp.int32)
    sw = ew.reshape(-1)[perm]
    counts = jnp.zeros((NE,), jnp.int32).at[ids_flat].add(1)
    padded = ((counts + BLK - 1) // BLK) * BLK
    pstart = jnp.concatenate(
        [jnp.zeros((1,), jnp.int32), jnp.cumsum(padded)[:-1].astype(jnp.int32)])
    gstart = jnp.concatenate(
        [jnp.zeros((1,), jnp.int32), jnp.cumsum(counts)[:-1].astype(jnp.int32)])
    i = jnp.arange(NPAIR, dtype=jnp.int32)
    ppos = pstart[se] + (i - gstart[se])
    gidx = jnp.zeros((PADTOT,), jnp.int32).at[ppos].set(stok)
    wrow = jnp.zeros((PADTOT, 1), jnp.float32).at[ppos, 0].set(sw)
    nb = padded // BLK
    cumnb = jnp.cumsum(nb)
    bexp = jnp.minimum(
        jnp.searchsorted(cumnb, jnp.arange(NBLK), side="right"), NE - 1
    ).astype(jnp.int32)
    pair_pos = jnp.zeros((NPAIR,), jnp.int32).at[perm].set(ppos)
    pp = pair_pos.reshape(T, K)
    cidx = jnp.concatenate([pp[:, 0], pp[:, 1]])
    return gidx, wrow, bexp, cidx


def _to_i32(a):
    n, m2 = a.shape
    return jax.lax.bitcast_convert_type(a.reshape(n, m2 // 2, 2), jnp.int32)


def _from_i32(a):
    n, m = a.shape
    return jax.lax.bitcast_convert_type(a, jnp.bfloat16).reshape(n, 2 * m)


def kernel(x, router_weight, router_bias, gate_up_proj, gate_up_proj_bias,
           down_proj, down_proj_bias):
    ids, ew = _router(x, router_weight, router_bias)
    gidx, wrow, bexp, cidx = _routing_metadata(ids, ew)
    xs = _from_i32(_sc_gather(_to_i32(x.astype(jnp.bfloat16)), gidx))
    ys = _grouped_mlp(bexp, xs, gate_up_proj, gate_up_proj_bias,
                      down_proj, down_proj_bias, wrow)
    g = _from_i32(_sc_gather(_to_i32(ys), cidx))
    out = _add_halves(g)
    return out, ew


# speedup vs baseline: 9.6076x; 2.1837x over previous
import jax
import jax.numpy as jnp
from jax.experimental import pallas as pl
from jax.experimental.pallas import tpu as pltpu

NE = 64
K = 2
D = 768
F = 768
T = 2048
NPAIR = T * K
BLK = 64
NBLK = NPAIR // BLK + NE
PADTOT = NBLK * BLK
CBLK = 256
ALPHA = 1.702
LIMIT = 7.0


def _router_body(x_ref, rw_ref, rb_ref, ids_ref, ew_ref):
    x = x_ref[...]
    rw = rw_ref[...]
    logits = jax.lax.dot_general(
        x, rw, (((1,), (1,)), ((), ())),
        preferred_element_type=jnp.float32,
        precision=jax.lax.Precision.DEFAULT,
    ) + rb_ref[...]
    iota = jax.lax.broadcasted_iota(jnp.int32, logits.shape, 1)
    m1 = jnp.max(logits, axis=1, keepdims=True)
    a1 = jnp.min(jnp.where(logits == m1, iota, NE), axis=1, keepdims=True)
    l2 = jnp.where(iota == a1, -jnp.inf, logits)
    m2 = jnp.max(l2, axis=1, keepdims=True)
    a2 = jnp.min(jnp.where(l2 == m2, iota, NE), axis=1, keepdims=True)
    z = jnp.exp(m2 - m1)
    w1 = 1.0 / (1.0 + z)
    w2 = z / (1.0 + z)
    ids_ref[...] = jnp.concatenate([a1, a2], axis=1)
    ew_ref[...] = jnp.concatenate([w1, w2], axis=1)


def _router(x, rw, rb, interpret=False):
    return pl.pallas_call(
        _router_body,
        out_shape=[
            jax.ShapeDtypeStruct((T, K), jnp.int32),
            jax.ShapeDtypeStruct((T, K), jnp.float32),
        ],
        interpret=interpret,
    )(x, rw, rb.reshape(1, NE))


def _mlp_body(be_ref, x_ref, gidx_ref, gup_ref, gupb_ref, dp_ref, dpb_ref,
              wrow_ref, sel_ref, ys_ref):
    gv = gidx_ref[0]
    tok = jax.lax.broadcasted_iota(jnp.int32, (BLK, T), 1)
    p = (gv == tok).astype(jnp.float32)
    x = jnp.dot(p, x_ref[...], preferred_element_type=jnp.float32)
    gu = jnp.dot(x, gup_ref[0], preferred_element_type=jnp.float32)
    gu = gu + gupb_ref[0]
    gmin = jnp.minimum(gu, LIMIT)
    glu = gmin * jax.nn.sigmoid(gmin * ALPHA)
    upc = jnp.clip(gu, -LIMIT, LIMIT) + 1.0
    up_on_gate = jnp.roll(upc, -1, axis=1)
    lane = jax.lax.broadcasted_iota(jnp.int32, gu.shape, 1)
    act2 = jnp.where(lane % 2 == 0, glu * up_on_gate, 0.0)
    act = jnp.dot(act2, sel_ref[...], preferred_element_type=jnp.float32)
    y = jnp.dot(act, dp_ref[0], preferred_element_type=jnp.float32)
    y = (y + dpb_ref[0]) * wrow_ref[...]
    ys_ref[...] = y.astype(jnp.bfloat16)


def _grouped_mlp(bexp, x, gidx, gup, gupb, dp, dpb, wrow, interpret=False):
    grid_spec = pltpu.PrefetchScalarGridSpec(
        num_scalar_prefetch=1,
        grid=(NBLK,),
        in_specs=[
            pl.BlockSpec((T, D), lambda b, be: (0, 0)),
            pl.BlockSpec((1, BLK, 1), lambda b, be: (b, 0, 0)),
            pl.BlockSpec((1, D, 2 * F), lambda b, be: (be[b], 0, 0)),
            pl.BlockSpec((1, 1, 2 * F), lambda b, be: (be[b], 0, 0)),
            pl.BlockSpec((1, F, D), lambda b, be: (be[b], 0, 0)),
            pl.BlockSpec((1, 1, D), lambda b, be: (be[b], 0, 0)),
            pl.BlockSpec((BLK, 1), lambda b, be: (b, 0)),
            pl.BlockSpec((2 * F, F), lambda b, be: (0, 0)),
        ],
        out_specs=pl.BlockSpec((BLK, D), lambda b, be: (b, 0)),
    )
    sel = (jnp.arange(2 * F, dtype=jnp.int32)[:, None]
           == 2 * jnp.arange(F, dtype=jnp.int32)[None, :]).astype(jnp.float32)
    return pl.pallas_call(
        _mlp_body,
        grid_spec=grid_spec,
        out_shape=jax.ShapeDtypeStruct((PADTOT, D), jnp.bfloat16),
        interpret=interpret,
    )(bexp, x, gidx.reshape(NBLK, BLK, 1), gup,
      gupb.reshape(NE, 1, 2 * F), dp, dpb.reshape(NE, 1, D), wrow, sel)


def _combine_body(i0_ref, i1_ref, ys_ref, o_ref):
    i0 = i0_ref[0]
    i1 = i1_ref[0]
    pos = jax.lax.broadcasted_iota(jnp.int32, (CBLK, PADTOT), 1)
    q = (i0 == pos).astype(jnp.float32) + (i1 == pos).astype(jnp.float32)
    o_ref[...] = jnp.dot(q, ys_ref[...].astype(jnp.float32),
                         preferred_element_type=jnp.float32)


def _combine(inv0, inv1, ys, interpret=False):
    nb = T // CBLK
    return pl.pallas_call(
        _combine_body,
        grid=(nb,),
        in_specs=[
            pl.BlockSpec((1, CBLK, 1), lambda i: (i, 0, 0)),
            pl.BlockSpec((1, CBLK, 1), lambda i: (i, 0, 0)),
            pl.BlockSpec((PADTOT, D), lambda i: (0, 0)),
        ],
        out_specs=pl.BlockSpec((CBLK, D), lambda i: (i, 0)),
        out_shape=jax.ShapeDtypeStruct((T, D), jnp.float32),
        interpret=interpret,
    )(inv0.reshape(nb, CBLK, 1), inv1.reshape(nb, CBLK, 1), ys)


def _routing_metadata(ids, ew):
    ids_flat = ids.reshape(-1)
    perm = jnp.argsort(ids_flat, stable=True).astype(jnp.int32)
    se = ids_flat[perm]
    stok = (perm // K).astype(jnp.int32)
    sw = ew.reshape(-1)[perm]
    counts = jnp.zeros((NE,), jnp.int32).at[ids_flat].add(1)
    padded = ((counts + BLK - 1) // BLK) * BLK
    pstart = jnp.concatenate(
        [jnp.zeros((1,), jnp.int32), jnp.cumsum(padded)[:-1].astype(jnp.int32)])
    gstart = jnp.concatenate(
        [jnp.zeros((1,), jnp.int32), jnp.cumsum(counts)[:-1].astype(jnp.int32)])
    i = jnp.arange(NPAIR, dtype=jnp.int32)
    ppos = pstart[se] + (i - gstart[se])
    gidx = jnp.zeros((PADTOT,), jnp.int32).at[ppos].set(stok)
    wrow = jnp.zeros((PADTOT, 1), jnp.float32).at[ppos, 0].set(sw)
    nb = padded // BLK
    cumnb = jnp.cumsum(nb)
    bexp = jnp.minimum(
        jnp.searchsorted(cumnb, jnp.arange(NBLK), side="right"), NE - 1
    ).astype(jnp.int32)
    pair_pos = jnp.zeros((NPAIR,), jnp.int32).at[perm].set(ppos)
    pp = pair_pos.reshape(T, K)
    return gidx, wrow, bexp, pp[:, 0], pp[:, 1]


def kernel(x, router_weight, router_bias, gate_up_proj, gate_up_proj_bias,
           down_proj, down_proj_bias):
    ids, ew = _router(x, router_weight, router_bias)
    gidx, wrow, bexp, inv0, inv1 = _routing_metadata(ids, ew)
    ys = _grouped_mlp(bexp, x, gidx, gate_up_proj, gate_up_proj_bias,
                      down_proj, down_proj_bias, wrow)
    out = _combine(inv0, inv1, ys)
    return out, ew
